# baseline (speedup 1.0000x reference)
import functools

import jax
import jax.numpy as jnp
from jax import lax
from jax.experimental import pallas as pl
from jax.experimental.pallas import tpu as pltpu
from jax.experimental.pallas import tpu_sc as plsc

_L = 16
_CW = 512
_CSH = 9
_NBUF = 2
_RING = 32


@functools.lru_cache(maxsize=None)
def _make_gather_kernel(B, D, V):
    info = plsc.get_sparse_core_info()
    NC, NS = info.num_cores, info.num_subcores
    NW = NC * NS
    n_full = V // _CW
    tail_w = V - n_full * _CW
    tail_c = n_full
    tail_owner = tail_c % NW

    mesh = plsc.VectorSubcoreMesh(core_axis_name="c", subcore_axis_name="s")

    @functools.partial(
        pl.kernel,
        mesh=mesh,
        compiler_params=pltpu.CompilerParams(needs_layout_passes=False),
        out_type=(jax.ShapeDtypeStruct((B * D,), jnp.float32),
                  jax.ShapeDtypeStruct((B * D,), jnp.float32)),
        scratch_types=[
            pltpu.VMEM((B,), jnp.int32),
            pltpu.VMEM((B,), jnp.int32),
            pltpu.VMEM((B,), jnp.int32),
            pltpu.VMEM((D, _CW), jnp.float32),
            pltpu.VMEM((D, _CW), jnp.float32),
            pltpu.VMEM((D, 64), jnp.float32),
            pltpu.VMEM((_RING * 64,), jnp.float32),
            pltpu.SMEM((64,), jnp.int32),
            pltpu.SMEM((64,), jnp.int32),
            pltpu.SemaphoreType.DMA,
            pltpu.SemaphoreType.DMA,
        ],
    )
    def k(users_hbm, items_hbm, uwT_hbm, iwT_hbm, ug_hbm, ig_hbm,
          idx_v, mu, mb, chunk0, chunk1, tailb,
          stage, offs_s, curs_s, sem_in, sem_out):
        wid = lax.axis_index("s") * NC + lax.axis_index("c")
        lane = lax.iota(jnp.int32, _L)
        bufs = (chunk0, chunk1)

        def process_table(tbl_hbm, src_idx_hbm, gout_hbm):
            for t in range(_NBUF - 1):
                pltpu.async_copy(
                    tbl_hbm.at[:, pl.ds((wid + NW * t) * _CW, _CW)],
                    bufs[t], sem_in)

            pltpu.sync_copy(src_idx_hbm, idx_v)

            def fbody(i, ptr):
                uvec = idx_v[pl.ds(i * _L, _L)]
                m = ((uvec >> _CSH) & (NW - 1)) == wid
                bvec = lane + i * _L
                plsc.store_compressed(mu.at[pl.ds(ptr, _L)], uvec, mask=m)
                plsc.store_compressed(mb.at[pl.ds(ptr, _L)], bvec, mask=m)
                return ptr + jnp.sum(jnp.where(m, 1, 0))

            M = lax.fori_loop(0, B // _L, fbody, 0, unroll=2)
            n_scan = (M + _L - 1) // _L
            jsh = _CSH + 5

            def hbody(g, cnts):
                jv = mu[pl.ds(g * _L, _L)] >> jsh
                jv = jnp.where((g * _L + lane) < M, jv, 63)
                for s in range(_L):
                    rot = jnp.take(jv, (lane + s) & (_L - 1))
                    cnts = tuple(
                        cq + jnp.where(rot == lane + 16 * q, 1, 0)
                        for q, cq in enumerate(cnts))
                return cnts

            zero = jnp.zeros((_L,), jnp.int32)
            cnts = lax.fori_loop(0, n_scan, hbody,
                                 (zero, zero, zero, zero))

            carry = 0
            for q in range(4):
                excl = plsc.cumsum(cnts[q]) - cnts[q] + carry
                for l in range(_L):
                    v = jnp.sum(jnp.where(lane == l, excl, 0))
                    offs_s[16 * q + l] = v
                    curs_s[16 * q + l] = v
                carry = carry + jnp.sum(cnts[q])

            def pbody(e, _):
                u_spl = plsc.load_gather(mu, [jnp.full((_L,), e, jnp.int32)])
                b_spl = plsc.load_gather(mb, [jnp.full((_L,), e, jnp.int32)])
                j_sc = jnp.max(u_spl) >> jsh
                p = curs_s[j_sc]
                curs_s[j_sc] = p + 1
                key = ((u_spl & (_CW - 1)) << 14) | b_spl
                plsc.store_scatter(idx_v, [jnp.full((_L,), p, jnp.int32)],
                                   key, mask=lane == 0)
                return 0

            lax.fori_loop(0, M, pbody, 0)

            def extract(j, buf, bufw, k0):
                lo = offs_s[j]
                hi = offs_s[j + 1]

                def ebody(e, ke):
                    r = ke & (_RING - 1)

                    @pl.when(ke >= _RING)
                    def _():
                        pltpu.make_async_copy(
                            stage.at[pl.ds(0, 64)],
                            gout_hbm.at[pl.ds(0, 64)], sem_out).wait()

                    key = plsc.load_gather(
                        idx_v, [jnp.full((_L,), e, jnp.int32)])
                    uoff = key >> 14
                    b_spl = key & 16383
                    for q in range(D // _L):
                        col = plsc.load_gather(buf, [lane + q * _L, uoff])
                        stage[pl.ds(r * 64 + q * _L, _L)] = col
                    b_sc = jnp.max(b_spl)
                    pltpu.async_copy(stage.at[pl.ds(r * 64, 64)],
                                     gout_hbm.at[pl.ds(b_sc * 64, 64)],
                                     sem_out)
                    return ke + 1

                return lax.fori_loop(lo, hi, ebody, k0)

            n_my = (n_full - 1 - wid) // NW + 1

            def _consume(j, t, kk):
                pltpu.make_async_copy(
                    tbl_hbm.at[:, pl.ds(0, _CW)], bufs[t], sem_in).wait()
                return extract(j, bufs[t], _CW, kk)

            def stream_grp(j4, kk):
                for t in range(_NBUF):
                    j = j4 * _NBUF + t

                    @pl.when(j + _NBUF - 1 < n_my)
                    def _():
                        c_nxt = wid + NW * (j + _NBUF - 1)
                        pltpu.async_copy(
                            tbl_hbm.at[:, pl.ds(c_nxt * _CW, _CW)],
                            bufs[(t + _NBUF - 1) % _NBUF], sem_in)

                    kk = lax.cond(
                        j < n_my,
                        lambda kk, jj=j, tt=t: _consume(jj, tt, kk),
                        lambda kk: kk,
                        kk)
                return kk

            k_tot = lax.fori_loop(
                0, (n_my + _NBUF - 1) // _NBUF, stream_grp, 0)

            @pl.when((wid == tail_owner) & (tail_w > 0))
            def _():
                pltpu.async_copy(
                    tbl_hbm.at[:, pl.ds(tail_c * _CW, tail_w)],
                    tailb, sem_in).wait()

            k_tot = lax.cond(
                (wid == tail_owner) & (tail_w > 0),
                lambda kk: extract(n_my, tailb, tail_w, kk),
                lambda kk: kk,
                k_tot)

            def dbody(i, _):
                pltpu.make_async_copy(stage.at[pl.ds(0, 64)],
                                      gout_hbm.at[pl.ds(0, 64)],
                                      sem_out).wait()
                return 0

            lax.fori_loop(0, jnp.minimum(k_tot, _RING), dbody, 0)

        process_table(uwT_hbm, users_hbm, ug_hbm)
        process_table(iwT_hbm, items_hbm, ig_hbm)

    return k


@functools.lru_cache(maxsize=None)
def _make_compute_kernel(B, D):
    info = plsc.get_sparse_core_info()
    NC, NS = info.num_cores, info.num_subcores
    NW = NC * NS
    b_per_w = B // NW
    n_dim_chunks = D // _L

    mesh = plsc.VectorSubcoreMesh(core_axis_name="c", subcore_axis_name="s")

    @functools.partial(
        pl.kernel,
        mesh=mesh,
        compiler_params=pltpu.CompilerParams(needs_layout_passes=False),
        out_type=jax.ShapeDtypeStruct((B,), jnp.float32),
        scratch_types=[
            pltpu.VMEM((b_per_w * D,), jnp.float32),
            pltpu.VMEM((b_per_w * D,), jnp.float32),
            pltpu.VMEM((b_per_w,), jnp.float32),
            pltpu.VMEM((_L * _L,), jnp.float32),
        ],
    )
    def k(ug_hbm, ig_hbm, out_hbm, uv, iv, res_v, trans):
        wid = lax.axis_index("s") * NC + lax.axis_index("c")
        base = wid * b_per_w
        pltpu.sync_copy(ug_hbm.at[pl.ds(base * D, b_per_w * D)], uv)
        pltpu.sync_copy(ig_hbm.at[pl.ds(base * D, b_per_w * D)], iv)
        lane16 = lax.iota(jnp.int32, _L) * _L

        def group_body(g, _):
            for r in range(_L):
                off = (g * _L + r) * D
                acc = jnp.zeros((_L,), jnp.float32)
                for q in range(n_dim_chunks):
                    u = uv[pl.ds(off + q * _L, _L)]
                    v = iv[pl.ds(off + q * _L, _L)]
                    acc = acc + (jnp.sign(u) + 1.0) * (jnp.sign(v) + 1.0)
                trans[pl.ds(r * _L, _L)] = acc
            res = plsc.load_gather(trans, [lane16])
            for c in range(1, _L):
                res = res + plsc.load_gather(trans, [lane16 + c])
            res = res * 0.25
            res_v[pl.ds(g * _L, _L)] = 1.0 / (1.0 + jnp.exp(1.0 - res))
            return 0

        lax.fori_loop(0, b_per_w // _L, group_body, 0)
        pltpu.sync_copy(res_v, out_hbm.at[pl.ds(base, b_per_w)])

    return k


def kernel(x, user_weight, item_weight):
    users = x[:, 0].astype(jnp.int32)
    items = x[:, 1].astype(jnp.int32)
    B = users.shape[0]
    V, D = user_weight.shape
    ka = _make_gather_kernel(B, D, V)
    ug, ig = ka(users, items, user_weight.T, item_weight.T)
    kb = _make_compute_kernel(B, D)
    return kb(ug, ig)


# speedup vs baseline: 1.0120x; 1.0120x over previous
import functools

import jax
import jax.numpy as jnp
from jax import lax
from jax.experimental import pallas as pl
from jax.experimental.pallas import tpu as pltpu
from jax.experimental.pallas import tpu_sc as plsc

_L = 16
_CW = 512
_CSH = 9
_NBUF = 2
_RING = 32


@functools.lru_cache(maxsize=None)
def _make_gather_kernel(B, D, V):
    info = plsc.get_sparse_core_info()
    NC, NS = info.num_cores, info.num_subcores
    NW = NC * NS
    n_full = V // _CW
    tail_w = V - n_full * _CW
    tail_c = n_full
    tail_owner = tail_c % NW

    mesh = plsc.VectorSubcoreMesh(core_axis_name="c", subcore_axis_name="s")

    @functools.partial(
        pl.kernel,
        mesh=mesh,
        compiler_params=pltpu.CompilerParams(needs_layout_passes=False),
        out_type=(jax.ShapeDtypeStruct((B * D,), jnp.float32),
                  jax.ShapeDtypeStruct((B * D,), jnp.float32)),
        scratch_types=[
            pltpu.VMEM((B,), jnp.int32),
            pltpu.VMEM((B,), jnp.int32),
            pltpu.VMEM((B,), jnp.int32),
            pltpu.VMEM((D, _CW), jnp.float32),
            pltpu.VMEM((D, _CW), jnp.float32),
            pltpu.VMEM((D, 64), jnp.float32),
            pltpu.VMEM((_RING * 64,), jnp.float32),
            pltpu.SMEM((64,), jnp.int32),
            pltpu.SMEM((64,), jnp.int32),
            pltpu.SemaphoreType.DMA,
            pltpu.SemaphoreType.DMA,
        ],
    )
    def k(users_hbm, items_hbm, uwT_hbm, iwT_hbm, ug_hbm, ig_hbm,
          idx_v, mu, mb, chunk0, chunk1, tailb,
          stage, offs_s, curs_s, sem_in, sem_out):
        wid = lax.axis_index("s") * NC + lax.axis_index("c")
        lane = lax.iota(jnp.int32, _L)
        bufs = (chunk0, chunk1)

        def process_table(tbl_hbm, src_idx_hbm, gout_hbm):
            for t in range(_NBUF - 1):
                pltpu.async_copy(
                    tbl_hbm.at[:, pl.ds((wid + NW * t) * _CW, _CW)],
                    bufs[t], sem_in)

            pltpu.sync_copy(src_idx_hbm, idx_v)

            def fbody(i, ptr):
                uvec = idx_v[pl.ds(i * _L, _L)]
                m = ((uvec >> _CSH) & (NW - 1)) == wid
                bvec = lane + i * _L
                plsc.store_compressed(mu.at[pl.ds(ptr, _L)], uvec, mask=m)
                plsc.store_compressed(mb.at[pl.ds(ptr, _L)], bvec, mask=m)
                return ptr + jnp.sum(jnp.where(m, 1, 0))

            M = lax.fori_loop(0, B // _L, fbody, 0)
            n_scan = (M + _L - 1) // _L
            jsh = _CSH + 5

            def hbody(g, cnts):
                jv = mu[pl.ds(g * _L, _L)] >> jsh
                jv = jnp.where((g * _L + lane) < M, jv, 63)
                for s in range(_L):
                    rot = jnp.take(jv, (lane + s) & (_L - 1))
                    cnts = tuple(
                        cq + jnp.where(rot == lane + 16 * q, 1, 0)
                        for q, cq in enumerate(cnts))
                return cnts

            zero = jnp.zeros((_L,), jnp.int32)
            cnts = lax.fori_loop(0, n_scan, hbody,
                                 (zero, zero, zero, zero))

            carry = 0
            for q in range(4):
                excl = plsc.cumsum(cnts[q]) - cnts[q] + carry
                for l in range(_L):
                    v = jnp.sum(jnp.where(lane == l, excl, 0))
                    offs_s[16 * q + l] = v
                    curs_s[16 * q + l] = v
                carry = carry + jnp.sum(cnts[q])

            def pbody(e, _):
                u_spl = plsc.load_gather(mu, [jnp.full((_L,), e, jnp.int32)])
                b_spl = plsc.load_gather(mb, [jnp.full((_L,), e, jnp.int32)])
                j_sc = jnp.max(u_spl) >> jsh
                p = curs_s[j_sc]
                curs_s[j_sc] = p + 1
                key = ((u_spl & (_CW - 1)) << 14) | b_spl
                plsc.store_scatter(idx_v, [jnp.full((_L,), p, jnp.int32)],
                                   key, mask=lane == 0)
                return 0

            lax.fori_loop(0, M, pbody, 0)

            def extract(j, buf, bufw, k0):
                lo = offs_s[j]
                hi = offs_s[j + 1]

                def ebody(e, ke):
                    r = ke & (_RING - 1)

                    @pl.when(ke >= _RING)
                    def _():
                        pltpu.make_async_copy(
                            stage.at[pl.ds(0, 64)],
                            gout_hbm.at[pl.ds(0, 64)], sem_out).wait()

                    key = plsc.load_gather(
                        idx_v, [jnp.full((_L,), e, jnp.int32)])
                    uoff = key >> 14
                    b_spl = key & 16383
                    for q in range(D // _L):
                        col = plsc.load_gather(buf, [lane + q * _L, uoff])
                        stage[pl.ds(r * 64 + q * _L, _L)] = col
                    b_sc = jnp.max(b_spl)
                    pltpu.async_copy(stage.at[pl.ds(r * 64, 64)],
                                     gout_hbm.at[pl.ds(b_sc * 64, 64)],
                                     sem_out)
                    return ke + 1

                return lax.fori_loop(lo, hi, ebody, k0)

            n_my = (n_full - 1 - wid) // NW + 1

            def _consume(j, t, kk):
                pltpu.make_async_copy(
                    tbl_hbm.at[:, pl.ds(0, _CW)], bufs[t], sem_in).wait()
                return extract(j, bufs[t], _CW, kk)

            def stream_grp(j4, kk):
                for t in range(_NBUF):
                    j = j4 * _NBUF + t

                    @pl.when(j + _NBUF - 1 < n_my)
                    def _():
                        c_nxt = wid + NW * (j + _NBUF - 1)
                        pltpu.async_copy(
                            tbl_hbm.at[:, pl.ds(c_nxt * _CW, _CW)],
                            bufs[(t + _NBUF - 1) % _NBUF], sem_in)

                    kk = lax.cond(
                        j < n_my,
                        lambda kk, jj=j, tt=t: _consume(jj, tt, kk),
                        lambda kk: kk,
                        kk)
                return kk

            k_tot = lax.fori_loop(
                0, (n_my + _NBUF - 1) // _NBUF, stream_grp, 0)

            @pl.when((wid == tail_owner) & (tail_w > 0))
            def _():
                pltpu.async_copy(
                    tbl_hbm.at[:, pl.ds(tail_c * _CW, tail_w)],
                    tailb, sem_in).wait()

            k_tot = lax.cond(
                (wid == tail_owner) & (tail_w > 0),
                lambda kk: extract(n_my, tailb, tail_w, kk),
                lambda kk: kk,
                k_tot)

            def dbody(i, _):
                pltpu.make_async_copy(stage.at[pl.ds(0, 64)],
                                      gout_hbm.at[pl.ds(0, 64)],
                                      sem_out).wait()
                return 0

            lax.fori_loop(0, jnp.minimum(k_tot, _RING), dbody, 0)

        process_table(uwT_hbm, users_hbm, ug_hbm)
        process_table(iwT_hbm, items_hbm, ig_hbm)

    return k


@functools.lru_cache(maxsize=None)
def _make_compute_kernel(B, D):
    info = plsc.get_sparse_core_info()
    NC, NS = info.num_cores, info.num_subcores
    NW = NC * NS
    b_per_w = B // NW
    n_dim_chunks = D // _L

    mesh = plsc.VectorSubcoreMesh(core_axis_name="c", subcore_axis_name="s")

    @functools.partial(
        pl.kernel,
        mesh=mesh,
        compiler_params=pltpu.CompilerParams(needs_layout_passes=False),
        out_type=jax.ShapeDtypeStruct((B,), jnp.float32),
        scratch_types=[
            pltpu.VMEM((b_per_w * D,), jnp.float32),
            pltpu.VMEM((b_per_w * D,), jnp.float32),
            pltpu.VMEM((b_per_w,), jnp.float32),
            pltpu.VMEM((_L * _L,), jnp.float32),
        ],
    )
    def k(ug_hbm, ig_hbm, out_hbm, uv, iv, res_v, trans):
        wid = lax.axis_index("s") * NC + lax.axis_index("c")
        base = wid * b_per_w
        pltpu.sync_copy(ug_hbm.at[pl.ds(base * D, b_per_w * D)], uv)
        pltpu.sync_copy(ig_hbm.at[pl.ds(base * D, b_per_w * D)], iv)
        lane16 = lax.iota(jnp.int32, _L) * _L

        def group_body(g, _):
            for r in range(_L):
                off = (g * _L + r) * D
                acc = jnp.zeros((_L,), jnp.float32)
                for q in range(n_dim_chunks):
                    u = uv[pl.ds(off + q * _L, _L)]
                    v = iv[pl.ds(off + q * _L, _L)]
                    acc = acc + (jnp.sign(u) + 1.0) * (jnp.sign(v) + 1.0)
                trans[pl.ds(r * _L, _L)] = acc
            res = plsc.load_gather(trans, [lane16])
            for c in range(1, _L):
                res = res + plsc.load_gather(trans, [lane16 + c])
            res = res * 0.25
            res_v[pl.ds(g * _L, _L)] = 1.0 / (1.0 + jnp.exp(1.0 - res))
            return 0

        lax.fori_loop(0, b_per_w // _L, group_body, 0)
        pltpu.sync_copy(res_v, out_hbm.at[pl.ds(base, b_per_w)])

    return k


def kernel(x, user_weight, item_weight):
    users = x[:, 0].astype(jnp.int32)
    items = x[:, 1].astype(jnp.int32)
    B = users.shape[0]
    V, D = user_weight.shape
    ka = _make_gather_kernel(B, D, V)
    ug, ig = ka(users, items, user_weight.T, item_weight.T)
    kb = _make_compute_kernel(B, D)
    return kb(ug, ig)
